# baseline (speedup 1.0000x reference)
import functools

import jax
import jax.numpy as jnp
from jax import lax
from jax.experimental import pallas as pl
from jax.experimental.pallas import tpu as pltpu
import jax.experimental.pallas.tpu_sc as plsc

E = 320000
N = 10000
D = 128
RAD = 64
H = 16
DH = 8

BE = 512
BN = 1000

NC = 2
NS = 16
NW = NC * NS
EPW = E // NW
CHUNK = 80
NCHUNK = EPW // CHUNK
NPAD = 10240
ROWS_PT = NPAD // NS


def _slrelu(x, a=0.2):
    return ((1.0 + a) / 2.0) * x + ((1.0 - a) / 2.0) * x * (
        2.0 * jax.nn.sigmoid(x) - 1.0)


def _edge_body(es_ref, msg_ref, ea_ref, w1_ref, b1_ref, w2_ref, b2_ref,
               wa_ref, ba_ref, wl_ref, bl_ref, wv_ref, bv_ref,
               g128_ref, attn_ref, p_ref):
    f32 = jnp.float32
    h = jax.nn.silu(
        jnp.dot(es_ref[...], w1_ref[...], preferred_element_type=f32)
        + b1_ref[...])
    weight = jnp.dot(h, w2_ref[...], preferred_element_type=f32) + b2_ref[...]
    ea = ea_ref[...]
    msg = msg_ref[...] * ea * weight
    la_act = _slrelu(
        jnp.dot(msg, wa_ref[...], preferred_element_type=f32) + ba_ref[...])
    la128 = jnp.dot(la_act, g128_ref[...], preferred_element_type=f32)
    p128 = jnp.exp(la128)
    value = jax.nn.silu(
        jnp.dot(msg, wl_ref[...], preferred_element_type=f32) + bl_ref[...])
    val = jnp.dot(value * ea, wv_ref[...], preferred_element_type=f32) \
        + bv_ref[...]
    attn_ref[...] = val * p128
    p_ref[...] = p128


def _edge_pipeline(message, edge_attr, edge_scalars,
                   W_rad1, b_rad1, W_rad2, b_rad2, W_alpha, b_alpha,
                   W_lin, b_lin, W_val, b_val, G128):
    full = lambda shape: pl.BlockSpec(shape, lambda i: (0, 0))
    grid = E // BE
    return pl.pallas_call(
        _edge_body,
        grid=(grid,),
        in_specs=[
            pl.BlockSpec((BE, RAD), lambda i: (i, 0)),
            pl.BlockSpec((BE, D), lambda i: (i, 0)),
            pl.BlockSpec((BE, 1), lambda i: (i, 0)),
            full((RAD, RAD)), full((1, RAD)),
            full((RAD, D)), full((1, D)),
            full((D, D)), full((1, D)),
            full((D, D)), full((1, D)),
            full((D, D)), full((1, D)),
            full((D, D)),
        ],
        out_specs=[
            pl.BlockSpec((BE, D), lambda i: (i, 0)),
            pl.BlockSpec((BE, D), lambda i: (i, 0)),
        ],
        out_shape=[
            jax.ShapeDtypeStruct((E, D), jnp.float32),
            jax.ShapeDtypeStruct((E, D), jnp.float32),
        ],
        compiler_params=pltpu.CompilerParams(
            dimension_semantics=("arbitrary",)),
    )(edge_scalars, message, edge_attr, W_rad1, b_rad1.reshape(1, RAD),
      W_rad2, b_rad2.reshape(1, D), W_alpha, b_alpha.reshape(1, D),
      W_lin, b_lin.reshape(1, D), W_val, b_val.reshape(1, D), G128)


def _sc_body(data_hbm, dst_hbm, out_hbm, acc,
             idx0, buf0, idx1, buf1, si0, sd0, si1, sd1):
    c = lax.axis_index("c")
    s = lax.axis_index("s")
    wid = c * NS + s

    def zrow(i, carry):
        def zcol(j, carry2):
            buf0[i, pl.ds(j * 16, 16)] = jnp.zeros((16,), jnp.float32)
            return carry2
        lax.fori_loop(0, D // 16, zcol, 0)
        return carry
    lax.fori_loop(0, CHUNK, zrow, 0)

    def zslab(k, carry):
        pltpu.sync_copy(buf0, acc.at[pl.ds(s * ROWS_PT + k * CHUNK, CHUNK), :])
        return carry
    lax.fori_loop(0, ROWS_PT // CHUNK, zslab, 0)
    plsc.subcore_barrier()

    base = wid * EPW

    def load(j, idxb, bufb, semi, semd):
        off = base + j * CHUNK
        pltpu.async_copy(dst_hbm.at[pl.ds(off, CHUNK)], idxb, semi)
        pltpu.async_copy(data_hbm.at[pl.ds(off, CHUNK), :], bufb, semd)

    def wait(j, idxb, bufb, semi, semd):
        off = base + j * CHUNK
        pltpu.make_async_copy(dst_hbm.at[pl.ds(off, CHUNK)], idxb, semi).wait()
        pltpu.make_async_copy(data_hbm.at[pl.ds(off, CHUNK), :], bufb,
                              semd).wait()

    load(0, idx0, buf0, si0, sd0)

    def pair(g, carry):
        j0 = 2 * g
        wait(j0, idx0, buf0, si0, sd0)
        load(j0 + 1, idx1, buf1, si1, sd1)
        pltpu.sync_copy(buf0, acc.at[idx0], add=True)
        wait(j0 + 1, idx1, buf1, si1, sd1)
        load(j0 + 2, idx0, buf0, si0, sd0)
        pltpu.sync_copy(buf1, acc.at[idx1], add=True)
        return carry

    lax.fori_loop(0, (NCHUNK - 1) // 2, pair, 0)
    jlast = NCHUNK - 1
    wait(jlast, idx0, buf0, si0, sd0)
    pltpu.sync_copy(buf0, acc.at[idx0], add=True)
    plsc.subcore_barrier()

    def dslab(k, carry):
        r = s * ROWS_PT + k * CHUNK
        pltpu.sync_copy(acc.at[pl.ds(r, CHUNK), :], buf0)
        pltpu.sync_copy(buf0, out_hbm.at[c, pl.ds(r, CHUNK), :])
        return carry
    lax.fori_loop(0, ROWS_PT // CHUNK, dslab, 0)


def _sc_scatter_one(data, dst32):
    mesh = plsc.VectorSubcoreMesh(core_axis_name="c", subcore_axis_name="s")
    f = pl.kernel(
        _sc_body,
        out_type=jax.ShapeDtypeStruct((NC, NPAD, D), jnp.float32),
        mesh=mesh,
        scratch_types=[
            pltpu.VMEM_SHARED((NPAD, D), jnp.float32),
            pltpu.VMEM((CHUNK,), jnp.int32),
            pltpu.VMEM((CHUNK, D), jnp.float32),
            pltpu.VMEM((CHUNK,), jnp.int32),
            pltpu.VMEM((CHUNK, D), jnp.float32),
            pltpu.SemaphoreType.DMA,
            pltpu.SemaphoreType.DMA,
            pltpu.SemaphoreType.DMA,
            pltpu.SemaphoreType.DMA,
        ],
        compiler_params=pltpu.CompilerParams(use_tc_tiling_on_sc=True),
    )
    return f(data, dst32)


@jax.jit
def _sc_scatter(attn, p, dst32):
    num2 = _sc_scatter_one(attn, dst32)
    p_b, num2 = lax.optimization_barrier((p, num2))
    den2 = _sc_scatter_one(p_b, dst32)
    return num2, den2


def _combine_body(num_ref, den_ref, wp_ref, bp_ref, out_ref):
    f32 = jnp.float32
    num = num_ref[0] + num_ref[1]
    den = den_ref[0] + den_ref[1]
    node = num / (den + 1e-16)
    out_ref[...] = jnp.dot(node, wp_ref[...], preferred_element_type=f32) \
        + bp_ref[...]


def _combine(num2, den2, W_proj, b_proj):
    full = lambda shape: pl.BlockSpec(shape, lambda i: (0, 0))
    return pl.pallas_call(
        _combine_body,
        grid=(N // BN,),
        in_specs=[
            pl.BlockSpec((NC, BN, D), lambda i: (0, i, 0)),
            pl.BlockSpec((NC, BN, D), lambda i: (0, i, 0)),
            full((D, D)), full((1, D)),
        ],
        out_specs=pl.BlockSpec((BN, D), lambda i: (i, 0)),
        out_shape=jax.ShapeDtypeStruct((N, D), jnp.float32),
        compiler_params=pltpu.CompilerParams(
            dimension_semantics=("arbitrary",)),
    )(num2, den2, W_proj, b_proj.reshape(1, D))


def kernel(message, edge_dst, edge_attr, edge_scalars, n_nodes_dst,
           W_rad1, b_rad1, W_rad2, b_rad2, W_alpha, b_alpha,
           W_lin, b_lin, W_val, b_val, alpha_dot, W_proj, b_proj):
    dst32 = edge_dst.astype(jnp.int32)

    heads = jnp.arange(D, dtype=jnp.int32) // DH
    same = (heads[:, None] == heads[None, :]).astype(jnp.float32)
    G128 = same * alpha_dot.reshape(D)[:, None]

    attn, p = _edge_pipeline(
        message, edge_attr, edge_scalars, W_rad1, b_rad1, W_rad2, b_rad2,
        W_alpha, b_alpha, W_lin, b_lin, W_val, b_val, G128)

    num2, den2 = _sc_scatter(attn, p, dst32)

    out = _combine(num2, den2, W_proj, b_proj)
    out = out + jnp.zeros((), dtype=out.dtype) * n_nodes_dst
    return out


# speedup vs baseline: 1.0940x; 1.0940x over previous
import functools

import jax
import jax.numpy as jnp
from jax import lax
from jax.experimental import pallas as pl
from jax.experimental.pallas import tpu as pltpu
import jax.experimental.pallas.tpu_sc as plsc

E = 320000
N = 10000
D = 128
RAD = 64
H = 16
DH = 8

BE = 512
BN = 1000

NC = 2
NS = 16
NW = NC * NS
EPW = E // NW
CHUNK = 80
NCHUNK = EPW // CHUNK
NPAD = 10240
ROWS_PT = NPAD // NS


def _slrelu(x, a=0.2):
    return ((1.0 + a) / 2.0) * x + ((1.0 - a) / 2.0) * x * (
        2.0 * jax.nn.sigmoid(x) - 1.0)


def _edge_body(es_ref, msg_ref, ea_ref, w1_ref, b1_ref, w2_ref, b2_ref,
               wa_ref, ba_ref, wl_ref, bl_ref, wv_ref, bv_ref,
               g128_ref, attn_ref, p_ref):
    f32 = jnp.float32
    h = jax.nn.silu(
        jnp.dot(es_ref[...], w1_ref[...], preferred_element_type=f32)
        + b1_ref[...])
    weight = jnp.dot(h, w2_ref[...], preferred_element_type=f32) + b2_ref[...]
    eaq = ea_ref[0]
    a = jnp.broadcast_to(eaq.reshape(BE // 128, 1, 128),
                         (BE // 128, 128, 128)).reshape(BE, 128)
    lane = lax.broadcasted_iota(jnp.int32, (BE, 128), 1)
    sub = lax.broadcasted_iota(jnp.int32, (BE, 128), 0)
    ea = jnp.sum(jnp.where(lane == sub % 128, a, 0.0), axis=1, keepdims=True)
    msg = msg_ref[...] * ea * weight
    la_act = _slrelu(
        jnp.dot(msg, wa_ref[...], preferred_element_type=f32) + ba_ref[...])
    la128 = jnp.dot(la_act, g128_ref[...], preferred_element_type=f32)
    p128 = jnp.exp(la128)
    value = jax.nn.silu(
        jnp.dot(msg, wl_ref[...], preferred_element_type=f32) + bl_ref[...])
    val = jnp.dot(value * ea, wv_ref[...], preferred_element_type=f32) \
        + bv_ref[...]
    attn_ref[...] = val * p128
    p_ref[...] = p128


def _edge_pipeline(message, edge_attr, edge_scalars,
                   W_rad1, b_rad1, W_rad2, b_rad2, W_alpha, b_alpha,
                   W_lin, b_lin, W_val, b_val, G128):
    full = lambda shape: pl.BlockSpec(shape, lambda i: (0, 0))
    grid = E // BE
    return pl.pallas_call(
        _edge_body,
        grid=(grid,),
        in_specs=[
            pl.BlockSpec((BE, RAD), lambda i: (i, 0)),
            pl.BlockSpec((BE, D), lambda i: (i, 0)),
            pl.BlockSpec((1, BE // 128, 128), lambda i: (i, 0, 0)),
            full((RAD, RAD)), full((1, RAD)),
            full((RAD, D)), full((1, D)),
            full((D, D)), full((1, D)),
            full((D, D)), full((1, D)),
            full((D, D)), full((1, D)),
            full((D, D)),
        ],
        out_specs=[
            pl.BlockSpec((BE, D), lambda i: (i, 0)),
            pl.BlockSpec((BE, D), lambda i: (i, 0)),
        ],
        out_shape=[
            jax.ShapeDtypeStruct((E, D), jnp.float32),
            jax.ShapeDtypeStruct((E, D), jnp.float32),
        ],
        compiler_params=pltpu.CompilerParams(
            dimension_semantics=("arbitrary",)),
    )(edge_scalars, message, edge_attr.reshape(E // BE, BE // 128, 128),
      W_rad1, b_rad1.reshape(1, RAD),
      W_rad2, b_rad2.reshape(1, D), W_alpha, b_alpha.reshape(1, D),
      W_lin, b_lin.reshape(1, D), W_val, b_val.reshape(1, D), G128)


def _sc_body(data_hbm, dst_hbm, out_hbm, acc,
             idx0, buf0, idx1, buf1, si0, sd0, si1, sd1):
    c = lax.axis_index("c")
    s = lax.axis_index("s")
    wid = c * NS + s

    def zrow(i, carry):
        def zcol(j, carry2):
            buf0[i, pl.ds(j * 16, 16)] = jnp.zeros((16,), jnp.float32)
            return carry2
        lax.fori_loop(0, D // 16, zcol, 0)
        return carry
    lax.fori_loop(0, CHUNK, zrow, 0)

    def zslab(k, carry):
        pltpu.sync_copy(buf0, acc.at[pl.ds(s * ROWS_PT + k * CHUNK, CHUNK), :])
        return carry
    lax.fori_loop(0, ROWS_PT // CHUNK, zslab, 0)
    plsc.subcore_barrier()

    base = wid * EPW

    def load(j, idxb, bufb, semi, semd):
        off = base + j * CHUNK
        pltpu.async_copy(dst_hbm.at[pl.ds(off, CHUNK)], idxb, semi)
        pltpu.async_copy(data_hbm.at[pl.ds(off, CHUNK), :], bufb, semd)

    def wait(j, idxb, bufb, semi, semd):
        off = base + j * CHUNK
        pltpu.make_async_copy(dst_hbm.at[pl.ds(off, CHUNK)], idxb, semi).wait()
        pltpu.make_async_copy(data_hbm.at[pl.ds(off, CHUNK), :], bufb,
                              semd).wait()

    load(0, idx0, buf0, si0, sd0)

    def pair(g, carry):
        j0 = 2 * g
        wait(j0, idx0, buf0, si0, sd0)
        load(j0 + 1, idx1, buf1, si1, sd1)
        pltpu.sync_copy(buf0, acc.at[idx0], add=True)
        wait(j0 + 1, idx1, buf1, si1, sd1)
        load(j0 + 2, idx0, buf0, si0, sd0)
        pltpu.sync_copy(buf1, acc.at[idx1], add=True)
        return carry

    lax.fori_loop(0, (NCHUNK - 1) // 2, pair, 0)
    jlast = NCHUNK - 1
    wait(jlast, idx0, buf0, si0, sd0)
    pltpu.sync_copy(buf0, acc.at[idx0], add=True)
    plsc.subcore_barrier()

    def dslab(k, carry):
        r = s * ROWS_PT + k * CHUNK
        pltpu.sync_copy(acc.at[pl.ds(r, CHUNK), :], buf0)
        pltpu.sync_copy(buf0, out_hbm.at[c, pl.ds(r, CHUNK), :])
        return carry
    lax.fori_loop(0, ROWS_PT // CHUNK, dslab, 0)


def _sc_scatter_one(data, dst32):
    mesh = plsc.VectorSubcoreMesh(core_axis_name="c", subcore_axis_name="s")
    f = pl.kernel(
        _sc_body,
        out_type=jax.ShapeDtypeStruct((NC, NPAD, D), jnp.float32),
        mesh=mesh,
        scratch_types=[
            pltpu.VMEM_SHARED((NPAD, D), jnp.float32),
            pltpu.VMEM((CHUNK,), jnp.int32),
            pltpu.VMEM((CHUNK, D), jnp.float32),
            pltpu.VMEM((CHUNK,), jnp.int32),
            pltpu.VMEM((CHUNK, D), jnp.float32),
            pltpu.SemaphoreType.DMA,
            pltpu.SemaphoreType.DMA,
            pltpu.SemaphoreType.DMA,
            pltpu.SemaphoreType.DMA,
        ],
        compiler_params=pltpu.CompilerParams(use_tc_tiling_on_sc=True),
    )
    return f(data, dst32)


@jax.jit
def _sc_scatter(attn, p, dst32):
    num2 = _sc_scatter_one(attn, dst32)
    p_b, num2 = lax.optimization_barrier((p, num2))
    den2 = _sc_scatter_one(p_b, dst32)
    return num2, den2


def _combine_body(num_ref, den_ref, wp_ref, bp_ref, out_ref):
    f32 = jnp.float32
    num = num_ref[0] + num_ref[1]
    den = den_ref[0] + den_ref[1]
    node = num / (den + 1e-16)
    out_ref[...] = jnp.dot(node, wp_ref[...], preferred_element_type=f32) \
        + bp_ref[...]


def _combine(num2, den2, W_proj, b_proj):
    full = lambda shape: pl.BlockSpec(shape, lambda i: (0, 0))
    return pl.pallas_call(
        _combine_body,
        grid=(N // BN,),
        in_specs=[
            pl.BlockSpec((NC, BN, D), lambda i: (0, i, 0)),
            pl.BlockSpec((NC, BN, D), lambda i: (0, i, 0)),
            full((D, D)), full((1, D)),
        ],
        out_specs=pl.BlockSpec((BN, D), lambda i: (i, 0)),
        out_shape=jax.ShapeDtypeStruct((N, D), jnp.float32),
        compiler_params=pltpu.CompilerParams(
            dimension_semantics=("arbitrary",)),
    )(num2, den2, W_proj, b_proj.reshape(1, D))


def kernel(message, edge_dst, edge_attr, edge_scalars, n_nodes_dst,
           W_rad1, b_rad1, W_rad2, b_rad2, W_alpha, b_alpha,
           W_lin, b_lin, W_val, b_val, alpha_dot, W_proj, b_proj):
    dst32 = edge_dst.astype(jnp.int32)

    heads = jnp.arange(D, dtype=jnp.int32) // DH
    same = (heads[:, None] == heads[None, :]).astype(jnp.float32)
    G128 = same * alpha_dot.reshape(D)[:, None]

    attn, p = _edge_pipeline(
        message, edge_attr, edge_scalars, W_rad1, b_rad1, W_rad2, b_rad2,
        W_alpha, b_alpha, W_lin, b_lin, W_val, b_val, G128)

    num2, den2 = _sc_scatter(attn, p, dst32)

    out = _combine(num2, den2, W_proj, b_proj)
    out = out + jnp.zeros((), dtype=out.dtype) * n_nodes_dst
    return out


# speedup vs baseline: 1.2799x; 1.1699x over previous
import functools

import jax
import jax.numpy as jnp
from jax import lax
from jax.experimental import pallas as pl
from jax.experimental.pallas import tpu as pltpu
import jax.experimental.pallas.tpu_sc as plsc

E = 320000
N = 10000
D = 128
RAD = 64
H = 16
DH = 8

BE = 640
BN = 1000

E2 = E // 2

NC = 2
NS = 16
NW = NC * NS
EPW = E2 // NW
CHUNK = 128
NFULL = EPW // CHUNK
TAIL = EPW - NFULL * CHUNK
NPAD = 10240
ROWS_PT = NPAD // NS


def _slrelu(x, a=0.2):
    return ((1.0 + a) / 2.0) * x + ((1.0 - a) / 2.0) * x * (
        2.0 * jax.nn.sigmoid(x) - 1.0)


def _edge_body(es_ref, msg_ref, ea_ref, w1_ref, b1_ref, w2_ref, b2_ref,
               wa_ref, ba_ref, wl_ref, bl_ref, wv_ref, bv_ref,
               g128_ref, attn_ref, p_ref):
    f32 = jnp.float32
    h = jax.nn.silu(
        jnp.dot(es_ref[...], w1_ref[...], preferred_element_type=f32)
        + b1_ref[...])
    weight = jnp.dot(h, w2_ref[...], preferred_element_type=f32) + b2_ref[...]
    eaq = ea_ref[0]
    a = jnp.broadcast_to(eaq.reshape(BE // 128, 1, 128),
                         (BE // 128, 128, 128)).reshape(BE, 128)
    lane = lax.broadcasted_iota(jnp.int32, (BE, 128), 1)
    sub = lax.broadcasted_iota(jnp.int32, (BE, 128), 0)
    ea = jnp.sum(jnp.where(lane == sub % 128, a, 0.0), axis=1, keepdims=True)
    msg = msg_ref[...] * ea * weight
    la_act = _slrelu(
        jnp.dot(msg, wa_ref[...], preferred_element_type=f32) + ba_ref[...])
    la128 = jnp.dot(la_act, g128_ref[...], preferred_element_type=f32)
    p128 = jnp.exp(la128)
    value = jax.nn.silu(
        jnp.dot(msg, wl_ref[...], preferred_element_type=f32) + bl_ref[...])
    val = jnp.dot(value * ea, wv_ref[...], preferred_element_type=f32) \
        + bv_ref[...]
    attn_ref[...] = val * p128
    p_ref[...] = p128


def _edge_pipeline(message, ea3, edge_scalars,
                   W_rad1, b_rad1, W_rad2, b_rad2, W_alpha, b_alpha,
                   W_lin, b_lin, W_val, b_val, G128, half):
    full = lambda shape: pl.BlockSpec(shape, lambda i: (0, 0))
    grid = E2 // BE
    off = half * grid
    return pl.pallas_call(
        _edge_body,
        grid=(grid,),
        in_specs=[
            pl.BlockSpec((BE, RAD), lambda i: (i + off, 0)),
            pl.BlockSpec((BE, D), lambda i: (i + off, 0)),
            pl.BlockSpec((1, BE // 128, 128), lambda i: (i + off, 0, 0)),
            full((RAD, RAD)), full((1, RAD)),
            full((RAD, D)), full((1, D)),
            full((D, D)), full((1, D)),
            full((D, D)), full((1, D)),
            full((D, D)), full((1, D)),
            full((D, D)),
        ],
        out_specs=[
            pl.BlockSpec((BE, D), lambda i: (i, 0)),
            pl.BlockSpec((BE, D), lambda i: (i, 0)),
        ],
        out_shape=[
            jax.ShapeDtypeStruct((E2, D), jnp.float32),
            jax.ShapeDtypeStruct((E2, D), jnp.float32),
        ],
        compiler_params=pltpu.CompilerParams(
            dimension_semantics=("arbitrary",)),
    )(edge_scalars, message, ea3, W_rad1, b_rad1.reshape(1, RAD),
      W_rad2, b_rad2.reshape(1, D), W_alpha, b_alpha.reshape(1, D),
      W_lin, b_lin.reshape(1, D), W_val, b_val.reshape(1, D), G128)


def _sc_body(half, data_hbm, dst_hbm, out_hbm, acc,
             idx0, buf0, idx1, buf1, idxt, buft, si0, sd0, si1, sd1):
    c = lax.axis_index("c")
    s = lax.axis_index("s")
    wid = c * NS + s

    def zrow(i, carry):
        def zcol(j, carry2):
            buf0[i, pl.ds(j * 16, 16)] = jnp.zeros((16,), jnp.float32)
            return carry2
        lax.fori_loop(0, D // 16, zcol, 0)
        return carry
    lax.fori_loop(0, CHUNK, zrow, 0)

    def zslab(k, carry):
        pltpu.sync_copy(buf0, acc.at[pl.ds(s * ROWS_PT + k * CHUNK, CHUNK), :])
        return carry
    lax.fori_loop(0, ROWS_PT // CHUNK, zslab, 0)
    plsc.subcore_barrier()

    base = wid * EPW
    dbase = half * E2 + base

    def load(j, idxb, bufb, semi, semd):
        pltpu.async_copy(dst_hbm.at[pl.ds(dbase + j * CHUNK, CHUNK)],
                         idxb, semi)
        pltpu.async_copy(data_hbm.at[pl.ds(base + j * CHUNK, CHUNK), :],
                         bufb, semd)

    def wait(j, idxb, bufb, semi, semd):
        pltpu.make_async_copy(dst_hbm.at[pl.ds(dbase + j * CHUNK, CHUNK)],
                              idxb, semi).wait()
        pltpu.make_async_copy(data_hbm.at[pl.ds(base + j * CHUNK, CHUNK), :],
                              bufb, semd).wait()

    load(0, idx0, buf0, si0, sd0)

    def pair(g, carry):
        j0 = 2 * g
        wait(j0, idx0, buf0, si0, sd0)
        load(j0 + 1, idx1, buf1, si1, sd1)
        pltpu.sync_copy(buf0, acc.at[idx0], add=True)
        wait(j0 + 1, idx1, buf1, si1, sd1)
        load(j0 + 2, idx0, buf0, si0, sd0)
        pltpu.sync_copy(buf1, acc.at[idx1], add=True)
        return carry

    lax.fori_loop(0, (NFULL - 1) // 2, pair, 0)
    jlast = NFULL - 1
    wait(jlast, idx0, buf0, si0, sd0)
    pltpu.sync_copy(buf0, acc.at[idx0], add=True)
    pltpu.sync_copy(dst_hbm.at[pl.ds(dbase + NFULL * CHUNK, TAIL)], idxt)
    pltpu.sync_copy(data_hbm.at[pl.ds(base + NFULL * CHUNK, TAIL), :], buft)
    pltpu.sync_copy(buft, acc.at[idxt], add=True)
    plsc.subcore_barrier()

    def dslab(k, carry):
        r = s * ROWS_PT + k * CHUNK
        pltpu.sync_copy(acc.at[pl.ds(r, CHUNK), :], buf0)
        pltpu.sync_copy(buf0, out_hbm.at[c, pl.ds(r, CHUNK), :])
        return carry
    lax.fori_loop(0, ROWS_PT // CHUNK, dslab, 0)


def _sc_scatter_one(data, dst32, half):
    mesh = plsc.VectorSubcoreMesh(core_axis_name="c", subcore_axis_name="s")
    f = pl.kernel(
        functools.partial(_sc_body, half),
        out_type=jax.ShapeDtypeStruct((NC, NPAD, D), jnp.float32),
        mesh=mesh,
        scratch_types=[
            pltpu.VMEM_SHARED((NPAD, D), jnp.float32),
            pltpu.VMEM((CHUNK,), jnp.int32),
            pltpu.VMEM((CHUNK, D), jnp.float32),
            pltpu.VMEM((CHUNK,), jnp.int32),
            pltpu.VMEM((CHUNK, D), jnp.float32),
            pltpu.VMEM((TAIL,), jnp.int32),
            pltpu.VMEM((TAIL, D), jnp.float32),
            pltpu.SemaphoreType.DMA,
            pltpu.SemaphoreType.DMA,
            pltpu.SemaphoreType.DMA,
            pltpu.SemaphoreType.DMA,
        ],
        compiler_params=pltpu.CompilerParams(use_tc_tiling_on_sc=True),
    )
    return f(data, dst32)


def _combine_body(na_ref, da_ref, nb_ref, db_ref, wp_ref, bp_ref, out_ref):
    f32 = jnp.float32
    num = (na_ref[0] + na_ref[1]) + (nb_ref[0] + nb_ref[1])
    den = (da_ref[0] + da_ref[1]) + (db_ref[0] + db_ref[1])
    node = num / (den + 1e-16)
    out_ref[...] = jnp.dot(node, wp_ref[...], preferred_element_type=f32) \
        + bp_ref[...]


def _combine(num_a, den_a, num_b, den_b, W_proj, b_proj):
    full = lambda shape: pl.BlockSpec(shape, lambda i: (0, 0))
    part = pl.BlockSpec((NC, BN, D), lambda i: (0, i, 0))
    return pl.pallas_call(
        _combine_body,
        grid=(N // BN,),
        in_specs=[part, part, part, part, full((D, D)), full((1, D))],
        out_specs=pl.BlockSpec((BN, D), lambda i: (i, 0)),
        out_shape=jax.ShapeDtypeStruct((N, D), jnp.float32),
        compiler_params=pltpu.CompilerParams(
            dimension_semantics=("arbitrary",)),
    )(num_a, den_a, num_b, den_b, W_proj, b_proj.reshape(1, D))


def kernel(message, edge_dst, edge_attr, edge_scalars, n_nodes_dst,
           W_rad1, b_rad1, W_rad2, b_rad2, W_alpha, b_alpha,
           W_lin, b_lin, W_val, b_val, alpha_dot, W_proj, b_proj):
    dst32 = edge_dst.astype(jnp.int32)

    heads = jnp.arange(D, dtype=jnp.int32) // DH
    same = (heads[:, None] == heads[None, :]).astype(jnp.float32)
    G128 = same * alpha_dot.reshape(D)[:, None]
    ea3 = edge_attr.reshape(E // BE, BE // 128, 128)

    def edge_half(h):
        return _edge_pipeline(
            message, ea3, edge_scalars, W_rad1, b_rad1, W_rad2, b_rad2,
            W_alpha, b_alpha, W_lin, b_lin, W_val, b_val, G128, h)

    attn0, p0 = edge_half(0)
    attn1, p1 = edge_half(1)

    num_a = _sc_scatter_one(attn0, dst32, 0)
    p0b, num_a = lax.optimization_barrier((p0, num_a))
    den_a = _sc_scatter_one(p0b, dst32, 0)
    attn1b, den_a = lax.optimization_barrier((attn1, den_a))
    num_b = _sc_scatter_one(attn1b, dst32, 1)
    p1b, num_b = lax.optimization_barrier((p1, num_b))
    den_b = _sc_scatter_one(p1b, dst32, 1)

    out = _combine(num_a, den_a, num_b, den_b, W_proj, b_proj)
    out = out + jnp.zeros((), dtype=out.dtype) * n_nodes_dst
    return out
